# baseline (speedup 1.0000x reference)
import functools

import jax
import jax.numpy as jnp
from jax import lax
from jax.experimental import pallas as pl
from jax.experimental.pallas import tpu as pltpu
from jax.experimental.pallas import tpu_sc as plsc

D = 32
DP = D // 2
B = 4096
NC = 2
NS = 16
L = 16
NW = NC * NS
BPW = B // NW

_mesh = plsc.VectorSubcoreMesh(core_axis_name="c", subcore_axis_name="s")


@functools.partial(
    pl.kernel,
    mesh=_mesh,
    out_type=jax.ShapeDtypeStruct((B,), jnp.float32),
    scratch_types=[
        pltpu.VMEM((BPW,), jnp.int32),
        pltpu.VMEM((BPW,), jnp.int32),
        pltpu.VMEM((BPW, DP), jnp.int32),
        pltpu.VMEM((BPW, DP), jnp.int32),
        pltpu.VMEM((BPW,), jnp.float32),
        pltpu.SemaphoreType.DMA,
    ],
    compiler_params=pltpu.CompilerParams(
        needs_layout_passes=False, use_tc_tiling_on_sc=False),
)
def _negdot(idxw_hbm, idxc_hbm, word_hbm, ctx_hbm, out_hbm,
            idxw_v, idxc_v, wrows_v, crows_v, out_v, sem):
    wid = lax.axis_index("s") * NC + lax.axis_index("c")
    base = wid * BPW

    pltpu.sync_copy(idxw_hbm.at[pl.ds(base, BPW)], idxw_v)
    pltpu.sync_copy(idxc_hbm.at[pl.ds(base, BPW)], idxc_v)

    cp_w = pltpu.async_copy(word_hbm.at[idxw_v], wrows_v, sem)
    cp_c = pltpu.async_copy(ctx_hbm.at[idxc_v], crows_v, sem)
    cp_w.wait()
    cp_c.wait()

    for g in range(BPW // L):
        rows = jnp.full((L,), g * L, jnp.int32) + lax.iota(jnp.int32, L)
        acc = jnp.zeros((L,), jnp.float32)
        for d2 in range(DP):
            col = jnp.full((L,), d2, jnp.int32)
            w_pk = plsc.load_gather(wrows_v, [rows, col])
            c_pk = plsc.load_gather(crows_v, [rows, col])
            wa, wb = plsc.unpack(plsc.bitcast(w_pk, jnp.bfloat16),
                                 format=plsc.PackFormat.INTERLEAVED)
            ca, cb = plsc.unpack(plsc.bitcast(c_pk, jnp.bfloat16),
                                 format=plsc.PackFormat.INTERLEAVED)
            acc = acc + wa * ca + wb * cb
        out_v[pl.ds(g * L, L)] = acc

    pltpu.sync_copy(out_v, out_hbm.at[pl.ds(base, BPW)])


def kernel(inputs, word_embeddings, context_embeddings):
    idx_word = inputs[:, 1].astype(jnp.int32)
    idx_ctx = inputs[:, 0].astype(jnp.int32)
    w_pk = lax.bitcast_convert_type(
        word_embeddings.astype(jnp.bfloat16).reshape(1000000, DP, 2),
        jnp.int32).reshape(1000000, DP)
    c_pk = lax.bitcast_convert_type(
        context_embeddings.astype(jnp.bfloat16).reshape(1000000, DP, 2),
        jnp.int32).reshape(1000000, DP)
    return _negdot(idx_word, idx_ctx, w_pk, c_pk)


# speedup vs baseline: 1.3903x; 1.3903x over previous
import functools

import jax
import jax.numpy as jnp
from jax import lax
from jax.experimental import pallas as pl
from jax.experimental.pallas import tpu as pltpu
from jax.experimental.pallas import tpu_sc as plsc

D = 32
R = 128
V = 1000000
B = 4096
NC = 2
NS = 16
L = 16
NW = NC * NS
BPW = B // NW

_mesh = plsc.VectorSubcoreMesh(core_axis_name="c", subcore_axis_name="s")


@functools.partial(
    pl.kernel,
    mesh=_mesh,
    out_type=jax.ShapeDtypeStruct((B,), jnp.float32),
    scratch_types=[
        pltpu.VMEM((BPW,), jnp.int32),
        pltpu.VMEM((BPW,), jnp.int32),
        pltpu.VMEM((BPW,), jnp.int32),
        pltpu.VMEM((BPW,), jnp.int32),
        pltpu.VMEM((BPW, R), jnp.float32),
        pltpu.VMEM((BPW, R), jnp.float32),
        pltpu.VMEM((BPW,), jnp.float32),
        pltpu.SemaphoreType.DMA,
    ],
    compiler_params=pltpu.CompilerParams(needs_layout_passes=False),
)
def _negdot(idxw_hbm, idxc_hbm, word_hbm, ctx_hbm, out_hbm,
            idxw_v, idxc_v, rww_v, rwc_v, wrows_v, crows_v, out_v, sem):
    wid = lax.axis_index("s") * NC + lax.axis_index("c")
    base = wid * BPW

    pltpu.sync_copy(idxw_hbm.at[pl.ds(base, BPW)], idxw_v)
    pltpu.sync_copy(idxc_hbm.at[pl.ds(base, BPW)], idxc_v)

    for j in range(BPW // L):
        rww_v[pl.ds(j * L, L)] = lax.shift_right_logical(
            idxw_v[pl.ds(j * L, L)], jnp.full((L,), 2, jnp.int32))
        rwc_v[pl.ds(j * L, L)] = lax.shift_right_logical(
            idxc_v[pl.ds(j * L, L)], jnp.full((L,), 2, jnp.int32))

    cp_w = pltpu.async_copy(word_hbm.at[rww_v], wrows_v, sem)
    cp_c = pltpu.async_copy(ctx_hbm.at[rwc_v], crows_v, sem)
    cp_w.wait()
    cp_c.wait()

    three = jnp.full((L,), 3, jnp.int32)
    five = jnp.full((L,), 5, jnp.int32)
    for g in range(BPW // L):
        rows = jnp.full((L,), g * L, jnp.int32) + lax.iota(jnp.int32, L)
        aw = lax.shift_left(idxw_v[pl.ds(g * L, L)] & three, five)
        ac = lax.shift_left(idxc_v[pl.ds(g * L, L)] & three, five)
        acc = jnp.zeros((L,), jnp.float32)
        for d in range(D):
            dd = jnp.full((L,), d, jnp.int32)
            w = plsc.load_gather(wrows_v, [rows, aw + dd])
            c = plsc.load_gather(crows_v, [rows, ac + dd])
            acc = acc + w * c
        out_v[pl.ds(g * L, L)] = acc

    pltpu.sync_copy(out_v, out_hbm.at[pl.ds(base, BPW)])


def kernel(inputs, word_embeddings, context_embeddings):
    idx_word = inputs[:, 1].astype(jnp.int32)
    idx_ctx = inputs[:, 0].astype(jnp.int32)
    one = lax.optimization_barrier(jnp.float32(1.0))
    return _negdot(idx_word, idx_ctx,
                   (word_embeddings * one).reshape(V // 4, R),
                   (context_embeddings * one).reshape(V // 4, R))


# speedup vs baseline: 7.9723x; 5.7342x over previous
import functools

import jax
import jax.numpy as jnp
from jax import lax
from jax.experimental import pallas as pl
from jax.experimental.pallas import tpu as pltpu
from jax.experimental.pallas import tpu_sc as plsc

D = 32
V = 1000000
B = 4096
GT = 16
GRID = B // GT
PW = GRID * 128
NC = 2
NS = 16
L = 16
NW = NC * NS
BPW = B // NW
CPW = PW // NW


def _tc_body(idxw_s, idxc_s, *refs):
    w_refs = refs[:GT]
    c_refs = refs[GT:2 * GT]
    out_ref = refs[2 * GT]
    i = pl.program_id(0)
    lane = lax.broadcasted_iota(jnp.int32, (D, 128), 1)
    acc = jnp.zeros((D, 128), jnp.float32)
    for k in range(GT):
        s = i * GT + k
        colw = idxw_s[s] & 127
        colc = idxc_s[s] & 127
        wsel = jnp.sum(jnp.where(lane == colw, w_refs[k][...], 0.0),
                       axis=1, keepdims=True)
        csel = jnp.sum(jnp.where(lane == colc, c_refs[k][...], 0.0),
                       axis=1, keepdims=True)
        acc = jnp.where(lane == k, wsel * csel, acc)
    out_ref[...] = acc


def _w_map(k):
    return lambda i, sw, sc: (0, sw[i * GT + k] >> 7)


def _c_map(k):
    return lambda i, sw, sc: (0, sc[i * GT + k] >> 7)


_tc_gather = pl.pallas_call(
    _tc_body,
    grid_spec=pltpu.PrefetchScalarGridSpec(
        num_scalar_prefetch=2,
        grid=(GRID,),
        in_specs=(
            [pl.BlockSpec((D, 128), _w_map(k)) for k in range(GT)]
            + [pl.BlockSpec((D, 128), _c_map(k)) for k in range(GT)]
        ),
        out_specs=pl.BlockSpec((D, 128), lambda i, sw, sc: (0, i)),
    ),
    out_shape=jax.ShapeDtypeStruct((D, PW), jnp.float32),
    compiler_params=pltpu.CompilerParams(
        dimension_semantics=("arbitrary",)),
)

_mesh = plsc.VectorSubcoreMesh(core_axis_name="c", subcore_axis_name="s")


@functools.partial(
    pl.kernel,
    mesh=_mesh,
    out_type=jax.ShapeDtypeStruct((B,), jnp.float32),
    scratch_types=[
        pltpu.VMEM((D, CPW), jnp.float32),
        pltpu.VMEM((BPW,), jnp.float32),
    ],
    compiler_params=pltpu.CompilerParams(needs_layout_passes=False),
)
def _sc_reduce(prod_hbm, out_hbm, slab_v, out_v):
    wid = lax.axis_index("s") * NC + lax.axis_index("c")
    pltpu.sync_copy(prod_hbm.at[:, pl.ds(wid * CPW, CPW)], slab_v)
    for g in range(BPW // L):
        acc = jnp.zeros((L,), jnp.float32)
        for d in range(D):
            acc = acc + slab_v[d, pl.ds(g * 128, L)]
        out_v[pl.ds(g * L, L)] = acc
    pltpu.sync_copy(out_v, out_hbm.at[pl.ds(wid * BPW, BPW)])


def kernel(inputs, word_embeddings, context_embeddings):
    idx_word = inputs[:, 1].astype(jnp.int32)
    idx_ctx = inputs[:, 0].astype(jnp.int32)
    prods = _tc_gather(idx_word, idx_ctx,
                       *([word_embeddings.T] * GT),
                       *([context_embeddings.T] * GT))
    return _sc_reduce(prods)
